# baseline (speedup 1.0000x reference)
import functools

import jax
import jax.numpy as jnp
from jax import lax
from jax.experimental import pallas as pl
from jax.experimental.pallas import tpu as pltpu
from jax.experimental.pallas import tpu_sc as plsc

NC = 2
NS = 16
NW = NC * NS
CH = 128
D = 128
G = 64

_mesh = plsc.VectorSubcoreMesh(core_axis_name="c", subcore_axis_name="s")


def _make_sc_degree(NP, EP, NCHW):
    R = NP // NS

    @functools.partial(
        pl.kernel,
        out_type=jax.ShapeDtypeStruct((NC, NP), jnp.float32),
        mesh=_mesh,
        scratch_types=[
            pltpu.VMEM((CH,), jnp.int32),
            pltpu.VMEM((CH,), jnp.float32),
            pltpu.VMEM_SHARED((NP,), jnp.float32),
        ],
    )
    def deg_kernel(dstp, ones_hbm, out, dv, onesv, acc):
        c = lax.axis_index("c")
        s = lax.axis_index("s")
        w = c * NS + s
        pltpu.sync_copy(ones_hbm.at[pl.ds(s * R, R)], acc.at[pl.ds(s * R, R)])
        pltpu.sync_copy(ones_hbm.at[pl.ds(0, CH)], onesv)
        plsc.subcore_barrier()
        base = w * CH * NCHW

        @pl.loop(0, NCHW)
        def _(j):
            pltpu.sync_copy(dstp.at[pl.ds(base + j * CH, CH)], dv)
            pltpu.sync_copy(onesv, acc.at[dv], add=True)

        plsc.subcore_barrier()
        pltpu.sync_copy(acc.at[pl.ds(s * R, R)], out.at[c, pl.ds(s * R, R)])

    return deg_kernel


def _make_sc_prop(NP, F0, F1):
    R = NP // NS

    @functools.partial(
        pl.kernel,
        out_type=jax.ShapeDtypeStruct((NC, NP, D), jnp.float32),
        mesh=_mesh,
        scratch_types=[
            pltpu.VMEM((CH,), jnp.int32),
            pltpu.VMEM((CH,), jnp.int32),
            pltpu.VMEM((CH,), jnp.int32),
            pltpu.VMEM((CH,), jnp.int32),
            pltpu.VMEM((CH, D), jnp.float32),
            pltpu.VMEM((CH, D), jnp.float32),
            pltpu.VMEM_SHARED((NP, D), jnp.float32),
            pltpu.SemaphoreType.DMA,
            pltpu.SemaphoreType.DMA,
        ],
    )
    def prop_kernel(hs, srcp, dstp, out, s0, s1, d0, d1, r0, r1, acc, m0, m1):
        c = lax.axis_index("c")
        s = lax.axis_index("s")
        pltpu.sync_copy(hs.at[pl.ds(s * R, R)], acc.at[pl.ds(s * R, R)])
        plsc.subcore_barrier()
        nchw = jnp.where(c == 0, F0, F1)
        base = jnp.where(c == 0, s * F0, NS * F0 + s * F1) * CH

        pltpu.sync_copy(srcp.at[pl.ds(base, CH)], s0)
        pltpu.sync_copy(dstp.at[pl.ds(base, CH)], d0)
        pltpu.async_copy(hs.at[s0], r0, m0)

        @pl.loop(0, nchw, step=2)
        def _(j):
            o1 = base + (j + 1) * CH
            pltpu.sync_copy(srcp.at[pl.ds(o1, CH)], s1)
            pltpu.sync_copy(dstp.at[pl.ds(o1, CH)], d1)
            pltpu.async_copy(hs.at[s1], r1, m1)
            pltpu.make_async_copy(hs.at[s0], r0, m0).wait()
            pltpu.sync_copy(r0, acc.at[d0], add=True)

            @pl.when(j + 2 < nchw)
            def _():
                o2 = base + (j + 2) * CH
                pltpu.sync_copy(srcp.at[pl.ds(o2, CH)], s0)
                pltpu.sync_copy(dstp.at[pl.ds(o2, CH)], d0)
                pltpu.async_copy(hs.at[s0], r0, m0)

            pltpu.make_async_copy(hs.at[s1], r1, m1).wait()
            pltpu.sync_copy(r1, acc.at[d1], add=True)

        plsc.subcore_barrier()
        pltpu.sync_copy(acc.at[pl.ds(s * R, R)], out.at[c, pl.ds(s * R, R)])

    return prop_kernel


def _tc_first(degp, x_p, W1, NP, BM):
    nblk = NP // BM

    def body(deg_ref, x_ref, w_ref, out_ref):
        dg = deg_ref[...]
        dinv = lax.rsqrt(dg[0] + dg[1] - 1.0)
        h = jnp.dot(x_ref[...], w_ref[...], preferred_element_type=jnp.float32)
        out_ref[...] = dinv[:, None] * h

    return pl.pallas_call(
        body,
        grid=(nblk,),
        in_specs=[
            pl.BlockSpec((NC, BM), lambda i: (0, i)),
            pl.BlockSpec((BM, D), lambda i: (i, 0)),
            pl.BlockSpec((D, D), lambda i: (0, 0)),
        ],
        out_specs=pl.BlockSpec((BM, D), lambda i: (i, 0)),
        out_shape=jax.ShapeDtypeStruct((NP, D), jnp.float32),
    )(degp, x_p, W1)


def _tc_mid(a, hs_prev, degp, b_prev, W, relu, NP, BM):
    nblk = NP // BM

    def body(a_ref, hs_ref, deg_ref, b_ref, w_ref, out_ref):
        dg = deg_ref[...]
        dinv = lax.rsqrt(dg[0] + dg[1] - 1.0)
        av = a_ref[...]
        t = dinv[:, None] * (av[0] + av[1] - hs_ref[...]) + b_ref[...]
        if relu:
            t = jnp.maximum(t, 0.0)
        out_ref[...] = dinv[:, None] * jnp.dot(
            t, w_ref[...], preferred_element_type=jnp.float32)

    return pl.pallas_call(
        body,
        grid=(nblk,),
        in_specs=[
            pl.BlockSpec((NC, BM, D), lambda i: (0, i, 0)),
            pl.BlockSpec((BM, D), lambda i: (i, 0)),
            pl.BlockSpec((NC, BM), lambda i: (0, i)),
            pl.BlockSpec((1, D), lambda i: (0, 0)),
            pl.BlockSpec((D, D), lambda i: (0, 0)),
        ],
        out_specs=pl.BlockSpec((BM, D), lambda i: (i, 0)),
        out_shape=jax.ShapeDtypeStruct((NP, D), jnp.float32),
    )(a, hs_prev, degp, b_prev, W)


def _tc_pool(a, hs_prev, degp, b_prev, batch_row, NP, BM):
    nblk = NP // BM

    def body(a_ref, hs_ref, deg_ref, b_ref, bat_ref, out_ref, acc_s, acc_c):
        i = pl.program_id(0)
        dg = deg_ref[...]
        dinv = lax.rsqrt(dg[0] + dg[1] - 1.0)
        av = a_ref[...]
        x3 = dinv[:, None] * (av[0] + av[1] - hs_ref[...]) + b_ref[...]
        gid = lax.broadcasted_iota(jnp.int32, (G, 1), 0)
        pt = (bat_ref[...] == gid).astype(jnp.float32)
        part = jnp.dot(pt, x3, preferred_element_type=jnp.float32)
        cnt = jnp.broadcast_to(jnp.sum(pt, axis=1, keepdims=True), (G, D))

        @pl.when(i == 0)
        def _():
            acc_s[...] = part
            acc_c[...] = cnt

        @pl.when(i > 0)
        def _():
            acc_s[...] += part
            acc_c[...] += cnt

        @pl.when(i == nblk - 1)
        def _():
            out_ref[...] = acc_s[...] / jnp.maximum(acc_c[...], 1.0)

    return pl.pallas_call(
        body,
        grid=(nblk,),
        in_specs=[
            pl.BlockSpec((NC, BM, D), lambda i: (0, i, 0)),
            pl.BlockSpec((BM, D), lambda i: (i, 0)),
            pl.BlockSpec((NC, BM), lambda i: (0, i)),
            pl.BlockSpec((1, D), lambda i: (0, 0)),
            pl.BlockSpec((1, BM), lambda i: (0, i)),
        ],
        out_specs=pl.BlockSpec((G, D), lambda i: (0, 0)),
        out_shape=jax.ShapeDtypeStruct((G, D), jnp.float32),
        scratch_shapes=[
            pltpu.VMEM((G, D), jnp.float32),
            pltpu.VMEM((G, D), jnp.float32),
        ],
    )(a, hs_prev, degp, b_prev, batch_row)


def kernel(x, edge_index, batch, W1, b1, W2, b2, W3, b3):
    N = x.shape[0]
    E = edge_index.shape[1]
    NP = (N // 2048 + 1) * 2048
    BM = NP // NS
    cpp = -(-E // (NS * CH))
    F0 = max(4, 4 * round(0.8 * cpp / 4))
    F1 = max(4, 4 * (-(-(cpp - F0) // 4)))
    EP = NS * (F0 + F1) * CH
    NCHW = (F0 + F1) // 2

    x_p = jnp.pad(x, ((0, NP - N), (0, 0)))
    pad_idx = jnp.full((EP - E,), N, jnp.int32)
    srcp = jnp.concatenate([edge_index[0], pad_idx])
    dstp = jnp.concatenate([edge_index[1], pad_idx])
    ones_h = jnp.ones((NP,), jnp.float32)
    batch_row = jnp.pad(batch, (0, NP - N), constant_values=G).reshape(1, NP)
    b1r, b2r, b3r = b1.reshape(1, D), b2.reshape(1, D), b3.reshape(1, D)

    degp = _make_sc_degree(NP, EP, NCHW)(dstp, ones_h)
    prop = _make_sc_prop(NP, F0, F1)

    hs1 = _tc_first(degp, x_p, W1, NP, BM)
    a1 = prop(hs1, srcp, dstp)
    hs2 = _tc_mid(a1, hs1, degp, b1r, W2, True, NP, BM)
    a2 = prop(hs2, srcp, dstp)
    hs3 = _tc_mid(a2, hs2, degp, b2r, W3, False, NP, BM)
    a3 = prop(hs3, srcp, dstp)
    return _tc_pool(a3, hs3, degp, b3r, batch_row, NP, BM)


# speedup vs baseline: 1.8758x; 1.8758x over previous
import functools

import jax
import jax.numpy as jnp
from jax import lax
from jax.experimental import pallas as pl
from jax.experimental.pallas import tpu as pltpu
from jax.experimental.pallas import tpu_sc as plsc

NC = 2
NS = 16
NW = NC * NS
CH = 128
D = 128
G = 64

_mesh = plsc.VectorSubcoreMesh(core_axis_name="c", subcore_axis_name="s")


def _make_sc_degree(NP, EP, NCHW):
    R = NP // NS

    @functools.partial(
        pl.kernel,
        out_type=jax.ShapeDtypeStruct((NC, NP), jnp.float32),
        mesh=_mesh,
        scratch_types=[
            pltpu.VMEM((CH,), jnp.int32),
            pltpu.VMEM((CH,), jnp.float32),
            pltpu.VMEM_SHARED((NP,), jnp.float32),
        ],
    )
    def deg_kernel(dstp, ones_hbm, out, dv, onesv, acc):
        c = lax.axis_index("c")
        s = lax.axis_index("s")
        w = c * NS + s
        pltpu.sync_copy(ones_hbm.at[pl.ds(s * R, R)], acc.at[pl.ds(s * R, R)])
        pltpu.sync_copy(ones_hbm.at[pl.ds(0, CH)], onesv)
        plsc.subcore_barrier()
        base = w * CH * NCHW

        @pl.loop(0, NCHW)
        def _(j):
            pltpu.sync_copy(dstp.at[pl.ds(base + j * CH, CH)], dv)
            pltpu.sync_copy(onesv, acc.at[dv], add=True)

        plsc.subcore_barrier()
        pltpu.sync_copy(acc.at[pl.ds(s * R, R)], out.at[c, pl.ds(s * R, R)])

    return deg_kernel


def _make_sc_prop(NP, F0, F1):
    R = NP // NS

    @functools.partial(
        pl.kernel,
        out_type=jax.ShapeDtypeStruct((NC, NP, D), jnp.float32),
        mesh=_mesh,
        scratch_types=[
            pltpu.VMEM((CH,), jnp.int32),
            pltpu.VMEM((CH,), jnp.int32),
            pltpu.VMEM((CH,), jnp.int32),
            pltpu.VMEM((CH,), jnp.int32),
            pltpu.VMEM((CH, D), jnp.float32),
            pltpu.VMEM((CH, D), jnp.float32),
            pltpu.VMEM_SHARED((NP, D), jnp.float32),
            pltpu.SemaphoreType.DMA,
            pltpu.SemaphoreType.DMA,
        ],
    )
    def prop_kernel(hs, srcp, dstp, out, s0, s1, d0, d1, r0, r1, acc, m0, m1):
        c = lax.axis_index("c")
        s = lax.axis_index("s")
        pltpu.sync_copy(hs.at[pl.ds(s * R, R)], acc.at[pl.ds(s * R, R)])
        plsc.subcore_barrier()
        nchw = jnp.where(c == 0, F0, F1)
        base = jnp.where(c == 0, s * F0, NS * F0 + s * F1) * CH

        pltpu.sync_copy(srcp.at[pl.ds(base, CH)], s0)
        pltpu.sync_copy(dstp.at[pl.ds(base, CH)], d0)
        pltpu.async_copy(hs.at[s0], r0, m0)

        @pl.loop(0, nchw, step=2)
        def _(j):
            o1 = base + (j + 1) * CH
            pltpu.sync_copy(srcp.at[pl.ds(o1, CH)], s1)
            pltpu.sync_copy(dstp.at[pl.ds(o1, CH)], d1)
            pltpu.async_copy(hs.at[s1], r1, m1)
            pltpu.make_async_copy(hs.at[s0], r0, m0).wait()
            pltpu.sync_copy(r0, acc.at[d0], add=True)

            @pl.when(j + 2 < nchw)
            def _():
                o2 = base + (j + 2) * CH
                pltpu.sync_copy(srcp.at[pl.ds(o2, CH)], s0)
                pltpu.sync_copy(dstp.at[pl.ds(o2, CH)], d0)
                pltpu.async_copy(hs.at[s0], r0, m0)

            pltpu.make_async_copy(hs.at[s1], r1, m1).wait()
            pltpu.sync_copy(r1, acc.at[d1], add=True)

        plsc.subcore_barrier()
        pltpu.sync_copy(acc.at[pl.ds(s * R, R)], out.at[c, pl.ds(s * R, R)])

    return prop_kernel


def _tc_first(degp, x_p, W1, NP, BM):
    nblk = NP // BM

    def body(deg_ref, x_ref, w_ref, out_ref):
        dg = deg_ref[...]
        dinv = lax.rsqrt(dg[0] + dg[1] - 1.0)
        h = jnp.dot(x_ref[...], w_ref[...], preferred_element_type=jnp.float32)
        out_ref[...] = dinv[:, None] * h

    return pl.pallas_call(
        body,
        grid=(nblk,),
        in_specs=[
            pl.BlockSpec((NC, BM), lambda i: (0, i)),
            pl.BlockSpec((BM, D), lambda i: (i, 0)),
            pl.BlockSpec((D, D), lambda i: (0, 0)),
        ],
        out_specs=pl.BlockSpec((BM, D), lambda i: (i, 0)),
        out_shape=jax.ShapeDtypeStruct((NP, D), jnp.float32),
    )(degp, x_p, W1)


def _tc_mid(a, hs_prev, degp, b_prev, W, relu, NP, BM):
    nblk = NP // BM

    def body(a_ref, hs_ref, deg_ref, b_ref, w_ref, out_ref):
        dg = deg_ref[...]
        dinv = lax.rsqrt(dg[0] + dg[1] - 1.0)
        av = a_ref[...]
        t = dinv[:, None] * (av[0] + av[1] - hs_ref[...]) + b_ref[...]
        if relu:
            t = jnp.maximum(t, 0.0)
        out_ref[...] = dinv[:, None] * jnp.dot(
            t, w_ref[...], preferred_element_type=jnp.float32)

    return pl.pallas_call(
        body,
        grid=(nblk,),
        in_specs=[
            pl.BlockSpec((NC, BM, D), lambda i: (0, i, 0)),
            pl.BlockSpec((BM, D), lambda i: (i, 0)),
            pl.BlockSpec((NC, BM), lambda i: (0, i)),
            pl.BlockSpec((1, D), lambda i: (0, 0)),
            pl.BlockSpec((D, D), lambda i: (0, 0)),
        ],
        out_specs=pl.BlockSpec((BM, D), lambda i: (i, 0)),
        out_shape=jax.ShapeDtypeStruct((NP, D), jnp.float32),
    )(a, hs_prev, degp, b_prev, W)


def _tc_pool(a, hs_prev, degp, b_prev, batch_row, NP, BM):
    nblk = NP // BM

    def body(a_ref, hs_ref, deg_ref, b_ref, bat_ref, out_ref, acc_s, acc_c):
        i = pl.program_id(0)
        dg = deg_ref[...]
        dinv = lax.rsqrt(dg[0] + dg[1] - 1.0)
        av = a_ref[...]
        x3 = dinv[:, None] * (av[0] + av[1] - hs_ref[...]) + b_ref[...]
        gid = lax.broadcasted_iota(jnp.int32, (G, 1), 0)
        pt = (bat_ref[...] == gid).astype(jnp.float32)
        part = jnp.dot(pt, x3, preferred_element_type=jnp.float32)
        cnt = jnp.broadcast_to(jnp.sum(pt, axis=1, keepdims=True), (G, D))

        @pl.when(i == 0)
        def _():
            acc_s[...] = part
            acc_c[...] = cnt

        @pl.when(i > 0)
        def _():
            acc_s[...] += part
            acc_c[...] += cnt

        @pl.when(i == nblk - 1)
        def _():
            out_ref[...] = acc_s[...] / jnp.maximum(acc_c[...], 1.0)

    return pl.pallas_call(
        body,
        grid=(nblk,),
        in_specs=[
            pl.BlockSpec((NC, BM, D), lambda i: (0, i, 0)),
            pl.BlockSpec((BM, D), lambda i: (i, 0)),
            pl.BlockSpec((NC, BM), lambda i: (0, i)),
            pl.BlockSpec((1, D), lambda i: (0, 0)),
            pl.BlockSpec((1, BM), lambda i: (0, i)),
        ],
        out_specs=pl.BlockSpec((G, D), lambda i: (0, 0)),
        out_shape=jax.ShapeDtypeStruct((G, D), jnp.float32),
        scratch_shapes=[
            pltpu.VMEM((G, D), jnp.float32),
            pltpu.VMEM((G, D), jnp.float32),
        ],
    )(a, hs_prev, degp, b_prev, batch_row)


def kernel(x, edge_index, batch, W1, b1, W2, b2, W3, b3):
    N = x.shape[0]
    E = edge_index.shape[1]
    NP = (N // 2048 + 1) * 2048
    BM = NP // NS
    cpp = -(-E // (NS * CH))
    F0 = max(2, 2 * round(0.8 * cpp / 2))
    F1 = max(2, 2 * (-(-(cpp - F0) // 2)))
    EP = NS * (F0 + F1) * CH
    NCHW = (F0 + F1) // 2

    x_p = jnp.pad(x, ((0, NP - N), (0, 0)))
    pad_idx = jnp.full((EP - E,), N, jnp.int32)
    srcp = jnp.concatenate([edge_index[0], pad_idx])
    dstp = jnp.concatenate([edge_index[1], pad_idx])
    ones_h = jnp.ones((NP,), jnp.float32)
    batch_row = jnp.pad(batch, (0, NP - N), constant_values=G).reshape(1, NP)
    b1r, b2r, b3r = b1.reshape(1, D), b2.reshape(1, D), b3.reshape(1, D)

    degp = _make_sc_degree(NP, EP, NCHW)(dstp, ones_h)
    prop = _make_sc_prop(NP, F0, F1)

    hs1 = _tc_first(degp, x_p, W1, NP, BM)
    a1 = prop(hs1, srcp, dstp)
    hs2 = _tc_mid(a1, hs1, degp, b1r, W2, True, NP, BM)
    a2 = prop(hs2, srcp, dstp)
    hs3 = _tc_mid(a2, hs2, degp, b2r, W3, False, NP, BM)
    a3 = prop(hs3, srcp, dstp)
    return _tc_pool(a3, hs3, degp, b3r, batch_row, NP, BM)


# speedup vs baseline: 2.8374x; 1.5127x over previous
import functools

import jax
import jax.numpy as jnp
from jax import lax
from jax.experimental import pallas as pl
from jax.experimental.pallas import tpu as pltpu
from jax.experimental.pallas import tpu_sc as plsc

NC = 2
NS = 16
NW = NC * NS
CH = 128
D = 128
G = 64

_mesh = plsc.VectorSubcoreMesh(core_axis_name="c", subcore_axis_name="s")


def _make_sc_degree(NP, EP, NCHW):
    R = NP // NS

    @functools.partial(
        pl.kernel,
        out_type=jax.ShapeDtypeStruct((NC, NP), jnp.float32),
        mesh=_mesh,
        scratch_types=[
            pltpu.VMEM((CH,), jnp.int32),
            pltpu.VMEM((CH,), jnp.float32),
            pltpu.VMEM_SHARED((NP,), jnp.float32),
        ],
    )
    def deg_kernel(dstp, ones_hbm, out, dv, onesv, acc):
        c = lax.axis_index("c")
        s = lax.axis_index("s")
        w = c * NS + s
        pltpu.sync_copy(ones_hbm.at[pl.ds(s * R, R)], acc.at[pl.ds(s * R, R)])
        pltpu.sync_copy(ones_hbm.at[pl.ds(0, CH)], onesv)
        plsc.subcore_barrier()
        base = w * CH * NCHW

        @pl.loop(0, NCHW)
        def _(j):
            pltpu.sync_copy(dstp.at[pl.ds(base + j * CH, CH)], dv)
            pltpu.sync_copy(onesv, acc.at[dv], add=True)

        plsc.subcore_barrier()
        pltpu.sync_copy(acc.at[pl.ds(s * R, R)], out.at[c, pl.ds(s * R, R)])

    return deg_kernel


def _make_sc_prop(NP, F0, F1):
    R = NP // NS

    @functools.partial(
        pl.kernel,
        out_type=jax.ShapeDtypeStruct((NC, NP, D), jnp.float32),
        mesh=_mesh,
        scratch_types=[
            pltpu.VMEM((CH,), jnp.int32),
            pltpu.VMEM((CH,), jnp.int32),
            pltpu.VMEM((CH,), jnp.int32),
            pltpu.VMEM((CH,), jnp.int32),
            pltpu.VMEM((CH, D), jnp.float32),
            pltpu.VMEM((CH, D), jnp.float32),
            pltpu.VMEM_SHARED((NP, D), jnp.float32),
            pltpu.SemaphoreType.DMA,
            pltpu.SemaphoreType.DMA,
        ],
    )
    def prop_kernel(hs, srcp, dstp, out, s0, s1, d0, d1, r0, r1, acc, m0, m1):
        c = lax.axis_index("c")
        s = lax.axis_index("s")
        pltpu.sync_copy(hs.at[pl.ds(s * R, R)], acc.at[pl.ds(s * R, R)])
        plsc.subcore_barrier()
        nchw = jnp.where(c == 0, F0, F1)
        base = jnp.where(c == 0, s * F0, NS * F0 + s * F1) * CH

        pltpu.sync_copy(srcp.at[pl.ds(base, CH)], s0)
        pltpu.sync_copy(dstp.at[pl.ds(base, CH)], d0)
        pltpu.async_copy(hs.at[s0], r0, m0)

        @pl.loop(0, nchw, step=2)
        def _(j):
            o1 = base + (j + 1) * CH
            pltpu.sync_copy(srcp.at[pl.ds(o1, CH)], s1)
            pltpu.sync_copy(dstp.at[pl.ds(o1, CH)], d1)
            pltpu.async_copy(hs.at[s1], r1, m1)
            pltpu.make_async_copy(hs.at[s0], r0, m0).wait()
            pltpu.sync_copy(r0, acc.at[d0], add=True)

            @pl.when(j + 2 < nchw)
            def _():
                o2 = base + (j + 2) * CH
                pltpu.sync_copy(srcp.at[pl.ds(o2, CH)], s0)
                pltpu.sync_copy(dstp.at[pl.ds(o2, CH)], d0)
                pltpu.async_copy(hs.at[s0], r0, m0)

            pltpu.make_async_copy(hs.at[s1], r1, m1).wait()
            pltpu.sync_copy(r1, acc.at[d1], add=True)

        plsc.subcore_barrier()
        pltpu.sync_copy(acc.at[pl.ds(s * R, R)], out.at[c, pl.ds(s * R, R)])

    return prop_kernel


def _tc_first(degp, x_p, W1, NP, BM):
    nblk = NP // BM

    def body(deg_ref, x_ref, w_ref, out_ref):
        dg = deg_ref[...]
        dinv = lax.rsqrt(dg[0] + dg[1] - 1.0)
        h = jnp.dot(x_ref[...], w_ref[...], preferred_element_type=jnp.float32)
        out_ref[...] = dinv[:, None] * h

    return pl.pallas_call(
        body,
        grid=(nblk,),
        in_specs=[
            pl.BlockSpec((NC, BM), lambda i: (0, i)),
            pl.BlockSpec((BM, D), lambda i: (i, 0)),
            pl.BlockSpec((D, D), lambda i: (0, 0)),
        ],
        out_specs=pl.BlockSpec((BM, D), lambda i: (i, 0)),
        out_shape=jax.ShapeDtypeStruct((NP, D), jnp.float32),
    )(degp, x_p, W1)


def _tc_mid(a, hs_prev, degp, b_prev, W, relu, NP, BM):
    nblk = NP // BM

    def body(a_ref, hs_ref, deg_ref, b_ref, w_ref, out_ref):
        dg = deg_ref[...]
        dinv = lax.rsqrt(dg[0] + dg[1] - 1.0)
        av = a_ref[...]
        t = dinv[:, None] * (av[0] + av[1] - hs_ref[...]) + b_ref[...]
        if relu:
            t = jnp.maximum(t, 0.0)
        out_ref[...] = dinv[:, None] * jnp.dot(
            t, w_ref[...], preferred_element_type=jnp.float32)

    return pl.pallas_call(
        body,
        grid=(nblk,),
        in_specs=[
            pl.BlockSpec((NC, BM, D), lambda i: (0, i, 0)),
            pl.BlockSpec((BM, D), lambda i: (i, 0)),
            pl.BlockSpec((NC, BM), lambda i: (0, i)),
            pl.BlockSpec((1, D), lambda i: (0, 0)),
            pl.BlockSpec((D, D), lambda i: (0, 0)),
        ],
        out_specs=pl.BlockSpec((BM, D), lambda i: (i, 0)),
        out_shape=jax.ShapeDtypeStruct((NP, D), jnp.float32),
    )(a, hs_prev, degp, b_prev, W)


def _tc_pool(a, hs_prev, degp, b_prev, batch_row, NP, BM):
    nblk = NP // BM

    def body(a_ref, hs_ref, deg_ref, b_ref, bat_ref, out_ref, acc_s, acc_c):
        i = pl.program_id(0)
        dg = deg_ref[...]
        dinv = lax.rsqrt(dg[0] + dg[1] - 1.0)
        av = a_ref[...]
        x3 = dinv[:, None] * (av[0] + av[1] - hs_ref[...]) + b_ref[...]
        gid = lax.broadcasted_iota(jnp.int32, (G, 1), 0)
        pt = (bat_ref[...] == gid).astype(jnp.float32)
        part = jnp.dot(pt, x3, preferred_element_type=jnp.float32)
        cnt = jnp.broadcast_to(jnp.sum(pt, axis=1, keepdims=True), (G, D))

        @pl.when(i == 0)
        def _():
            acc_s[...] = part
            acc_c[...] = cnt

        @pl.when(i > 0)
        def _():
            acc_s[...] += part
            acc_c[...] += cnt

        @pl.when(i == nblk - 1)
        def _():
            out_ref[...] = acc_s[...] / jnp.maximum(acc_c[...], 1.0)

    return pl.pallas_call(
        body,
        grid=(nblk,),
        in_specs=[
            pl.BlockSpec((NC, BM, D), lambda i: (0, i, 0)),
            pl.BlockSpec((BM, D), lambda i: (i, 0)),
            pl.BlockSpec((NC, BM), lambda i: (0, i)),
            pl.BlockSpec((1, D), lambda i: (0, 0)),
            pl.BlockSpec((1, BM), lambda i: (0, i)),
        ],
        out_specs=pl.BlockSpec((G, D), lambda i: (0, 0)),
        out_shape=jax.ShapeDtypeStruct((G, D), jnp.float32),
        scratch_shapes=[
            pltpu.VMEM((G, D), jnp.float32),
            pltpu.VMEM((G, D), jnp.float32),
        ],
    )(a, hs_prev, degp, b_prev, batch_row)


def kernel(x, edge_index, batch, W1, b1, W2, b2, W3, b3):
    N = x.shape[0]
    E = edge_index.shape[1]
    NP = (N // 2048 + 1) * 2048
    BM = NP // NS
    cpp = -(-E // (NS * CH))
    F0 = max(2, 2 * round(0.5 * cpp / 2))
    F1 = max(2, 2 * (-(-(cpp - F0) // 2)))
    EP = NS * (F0 + F1) * CH
    NCHW = (F0 + F1) // 2

    x_p = jnp.pad(x, ((0, NP - N), (0, 0)))
    pad_idx = N + jnp.arange(EP - E, dtype=jnp.int32) % (NP - N)
    srcp = jnp.concatenate([edge_index[0], pad_idx])
    dstp = jnp.concatenate([edge_index[1], pad_idx])
    ones_h = jnp.ones((NP,), jnp.float32)
    batch_row = jnp.pad(batch, (0, NP - N), constant_values=G).reshape(1, NP)
    b1r, b2r, b3r = b1.reshape(1, D), b2.reshape(1, D), b3.reshape(1, D)

    degp = _make_sc_degree(NP, EP, NCHW)(dstp, ones_h)
    prop = _make_sc_prop(NP, F0, F1)

    hs1 = _tc_first(degp, x_p, W1, NP, BM)
    a1 = prop(hs1, srcp, dstp)
    hs2 = _tc_mid(a1, hs1, degp, b1r, W2, True, NP, BM)
    a2 = prop(hs2, srcp, dstp)
    hs3 = _tc_mid(a2, hs2, degp, b2r, W3, False, NP, BM)
    a3 = prop(hs3, srcp, dstp)
    return _tc_pool(a3, hs3, degp, b3r, batch_row, NP, BM)


# speedup vs baseline: 3.8003x; 1.3393x over previous
import functools

import jax
import jax.numpy as jnp
from jax import lax
from jax.experimental import pallas as pl
from jax.experimental.pallas import tpu as pltpu
from jax.experimental.pallas import tpu_sc as plsc

NC = 2
NS = 16
NW = NC * NS
CH = 128
D = 128
G = 64

_mesh = plsc.VectorSubcoreMesh(core_axis_name="c", subcore_axis_name="s")


def _make_sc_degree(NP, F):
    R = NP // NS

    @functools.partial(
        pl.kernel,
        out_type=jax.ShapeDtypeStruct((NC, NP), jnp.float32),
        mesh=_mesh,
        scratch_types=[
            pltpu.VMEM((F, CH), jnp.int32),
            pltpu.VMEM((CH,), jnp.float32),
            pltpu.VMEM_SHARED((NP,), jnp.float32),
        ],
    )
    def deg_kernel(dst2, ones_hbm, out, dbig, onesv, acc):
        c = lax.axis_index("c")
        s = lax.axis_index("s")
        w = c * NS + s
        pltpu.sync_copy(ones_hbm.at[pl.ds(s * R, R)], acc.at[pl.ds(s * R, R)])
        pltpu.sync_copy(ones_hbm.at[pl.ds(0, CH)], onesv)
        pltpu.sync_copy(dst2.at[pl.ds(w * F, F)], dbig)
        plsc.subcore_barrier()

        @pl.loop(0, F)
        def _(j):
            pltpu.sync_copy(onesv, acc.at[dbig.at[j]], add=True)

        plsc.subcore_barrier()
        pltpu.sync_copy(acc.at[pl.ds(s * R, R)], out.at[c, pl.ds(s * R, R)])

    return deg_kernel


def _make_sc_prop(NP, F):
    R = NP // NS
    PH = F // 2

    @functools.partial(
        pl.kernel,
        out_type=jax.ShapeDtypeStruct((NC, NP, D), jnp.float32),
        mesh=_mesh,
        scratch_types=[
            pltpu.VMEM((PH, CH), jnp.int32),
            pltpu.VMEM((PH, CH), jnp.int32),
            pltpu.VMEM((CH, D), jnp.float32),
            pltpu.VMEM((CH, D), jnp.float32),
            pltpu.VMEM_SHARED((NP, D), jnp.float32),
            pltpu.SemaphoreType.DMA,
            pltpu.SemaphoreType.DMA,
        ],
    )
    def prop_kernel(hs, src2, dst2, out, sbig, dbig, r0, r1, acc, m0, m1):
        c = lax.axis_index("c")
        s = lax.axis_index("s")
        w = c * NS + s
        pltpu.sync_copy(hs.at[pl.ds(s * R, R)], acc.at[pl.ds(s * R, R)])
        plsc.subcore_barrier()

        for p in range(2):
            row0 = w * F + p * PH
            pltpu.sync_copy(src2.at[pl.ds(row0, PH)], sbig)
            pltpu.sync_copy(dst2.at[pl.ds(row0, PH)], dbig)
            pltpu.async_copy(hs.at[sbig.at[0]], r0, m0)

            @pl.loop(0, PH, step=2)
            def _(j):
                @pl.when(j + 1 < PH)
                def _():
                    pltpu.async_copy(hs.at[sbig.at[j + 1]], r1, m1)

                pltpu.make_async_copy(hs.at[sbig.at[j]], r0, m0).wait()
                pltpu.sync_copy(r0, acc.at[dbig.at[j]], add=True)

                @pl.when(j + 2 < PH)
                def _():
                    pltpu.async_copy(hs.at[sbig.at[j + 2]], r0, m0)

                pltpu.make_async_copy(hs.at[sbig.at[j + 1]], r1, m1).wait()
                pltpu.sync_copy(r1, acc.at[dbig.at[j + 1]], add=True)

        plsc.subcore_barrier()
        pltpu.sync_copy(acc.at[pl.ds(s * R, R)], out.at[c, pl.ds(s * R, R)])

    return prop_kernel


def _tc_first(degp, x_p, W1, NP, BM):
    nblk = NP // BM

    def body(deg_ref, x_ref, w_ref, out_ref):
        dg = deg_ref[...]
        dinv = lax.rsqrt(dg[0] + dg[1] - 1.0)
        h = jnp.dot(x_ref[...], w_ref[...], preferred_element_type=jnp.float32)
        out_ref[...] = dinv[:, None] * h

    return pl.pallas_call(
        body,
        grid=(nblk,),
        in_specs=[
            pl.BlockSpec((NC, BM), lambda i: (0, i)),
            pl.BlockSpec((BM, D), lambda i: (i, 0)),
            pl.BlockSpec((D, D), lambda i: (0, 0)),
        ],
        out_specs=pl.BlockSpec((BM, D), lambda i: (i, 0)),
        out_shape=jax.ShapeDtypeStruct((NP, D), jnp.float32),
    )(degp, x_p, W1)


def _tc_mid(a, hs_prev, degp, b_prev, W, relu, NP, BM):
    nblk = NP // BM

    def body(a_ref, hs_ref, deg_ref, b_ref, w_ref, out_ref):
        dg = deg_ref[...]
        dinv = lax.rsqrt(dg[0] + dg[1] - 1.0)
        av = a_ref[...]
        t = dinv[:, None] * (av[0] + av[1] - hs_ref[...]) + b_ref[...]
        if relu:
            t = jnp.maximum(t, 0.0)
        out_ref[...] = dinv[:, None] * jnp.dot(
            t, w_ref[...], preferred_element_type=jnp.float32)

    return pl.pallas_call(
        body,
        grid=(nblk,),
        in_specs=[
            pl.BlockSpec((NC, BM, D), lambda i: (0, i, 0)),
            pl.BlockSpec((BM, D), lambda i: (i, 0)),
            pl.BlockSpec((NC, BM), lambda i: (0, i)),
            pl.BlockSpec((1, D), lambda i: (0, 0)),
            pl.BlockSpec((D, D), lambda i: (0, 0)),
        ],
        out_specs=pl.BlockSpec((BM, D), lambda i: (i, 0)),
        out_shape=jax.ShapeDtypeStruct((NP, D), jnp.float32),
    )(a, hs_prev, degp, b_prev, W)


def _tc_pool(a, hs_prev, degp, b_prev, batch_row, NP, BM):
    nblk = NP // BM

    def body(a_ref, hs_ref, deg_ref, b_ref, bat_ref, out_ref, acc_s, acc_c):
        i = pl.program_id(0)
        dg = deg_ref[...]
        dinv = lax.rsqrt(dg[0] + dg[1] - 1.0)
        av = a_ref[...]
        x3 = dinv[:, None] * (av[0] + av[1] - hs_ref[...]) + b_ref[...]
        gid = lax.broadcasted_iota(jnp.int32, (G, 1), 0)
        pt = (bat_ref[...] == gid).astype(jnp.float32)
        part = jnp.dot(pt, x3, preferred_element_type=jnp.float32)
        cnt = jnp.broadcast_to(jnp.sum(pt, axis=1, keepdims=True), (G, D))

        @pl.when(i == 0)
        def _():
            acc_s[...] = part
            acc_c[...] = cnt

        @pl.when(i > 0)
        def _():
            acc_s[...] += part
            acc_c[...] += cnt

        @pl.when(i == nblk - 1)
        def _():
            out_ref[...] = acc_s[...] / jnp.maximum(acc_c[...], 1.0)

    return pl.pallas_call(
        body,
        grid=(nblk,),
        in_specs=[
            pl.BlockSpec((NC, BM, D), lambda i: (0, i, 0)),
            pl.BlockSpec((BM, D), lambda i: (i, 0)),
            pl.BlockSpec((NC, BM), lambda i: (0, i)),
            pl.BlockSpec((1, D), lambda i: (0, 0)),
            pl.BlockSpec((1, BM), lambda i: (0, i)),
        ],
        out_specs=pl.BlockSpec((G, D), lambda i: (0, 0)),
        out_shape=jax.ShapeDtypeStruct((G, D), jnp.float32),
        scratch_shapes=[
            pltpu.VMEM((G, D), jnp.float32),
            pltpu.VMEM((G, D), jnp.float32),
        ],
    )(a, hs_prev, degp, b_prev, batch_row)


def kernel(x, edge_index, batch, W1, b1, W2, b2, W3, b3):
    N = x.shape[0]
    E = edge_index.shape[1]
    NP = (N // 2048 + 1) * 2048
    BM = NP // NS
    F = 4 * (-(-E // (NW * CH * 4)))
    EP = NW * F * CH

    x_p = jnp.pad(x, ((0, NP - N), (0, 0)))
    pad_idx = N + jnp.arange(EP - E, dtype=jnp.int32) % (NP - N)
    srcp = jnp.concatenate([edge_index[0], pad_idx])
    dstp = jnp.concatenate([edge_index[1], pad_idx])
    ones_h = jnp.ones((NP,), jnp.float32)
    batch_row = jnp.pad(batch, (0, NP - N), constant_values=G).reshape(1, NP)
    b1r, b2r, b3r = b1.reshape(1, D), b2.reshape(1, D), b3.reshape(1, D)

    src2 = srcp.reshape(EP // CH, CH)
    dst2 = dstp.reshape(EP // CH, CH)
    degp = _make_sc_degree(NP, F)(dst2, ones_h)
    prop = _make_sc_prop(NP, F)

    hs1 = _tc_first(degp, x_p, W1, NP, BM)
    a1 = prop(hs1, src2, dst2)
    hs2 = _tc_mid(a1, hs1, degp, b1r, W2, True, NP, BM)
    a2 = prop(hs2, src2, dst2)
    hs3 = _tc_mid(a2, hs2, degp, b2r, W3, False, NP, BM)
    a3 = prop(hs3, src2, dst2)
    return _tc_pool(a3, hs3, degp, b3r, batch_row, NP, BM)


# speedup vs baseline: 4.0049x; 1.0538x over previous
import functools

import jax
import jax.numpy as jnp
from jax import lax
from jax.experimental import pallas as pl
from jax.experimental.pallas import tpu as pltpu
from jax.experimental.pallas import tpu_sc as plsc

NC = 2
NS = 16
NW = NC * NS
CH = 128
D = 128
G = 64

_mesh = plsc.VectorSubcoreMesh(core_axis_name="c", subcore_axis_name="s")


def _make_sc_degree(NP, F):
    R = NP // NS

    @functools.partial(
        pl.kernel,
        out_type=jax.ShapeDtypeStruct((NC, NP), jnp.float32),
        mesh=_mesh,
        scratch_types=[
            pltpu.VMEM((F, 2, CH), jnp.int32),
            pltpu.VMEM((CH,), jnp.float32),
            pltpu.VMEM_SHARED((NP,), jnp.float32),
        ],
    )
    def deg_kernel(ei3, ones_hbm, out, ebig, onesv, acc):
        c = lax.axis_index("c")
        s = lax.axis_index("s")
        w = c * NS + s
        pltpu.sync_copy(ones_hbm.at[pl.ds(s * R, R)], acc.at[pl.ds(s * R, R)])
        pltpu.sync_copy(ones_hbm.at[pl.ds(0, CH)], onesv)
        pltpu.sync_copy(ei3.at[pl.ds(w * F, F)], ebig)
        plsc.subcore_barrier()

        @pl.loop(0, F)
        def _(j):
            pltpu.sync_copy(onesv, acc.at[ebig.at[j, 1]], add=True)

        plsc.subcore_barrier()
        pltpu.sync_copy(acc.at[pl.ds(s * R, R)], out.at[c, pl.ds(s * R, R)])

    return deg_kernel


def _make_sc_prop(NP, F):
    R = NP // NS
    PH = F // 2

    @functools.partial(
        pl.kernel,
        out_type=jax.ShapeDtypeStruct((NC, NP, D), jnp.float32),
        mesh=_mesh,
        scratch_types=[
            pltpu.VMEM((PH, 2, CH), jnp.int32),
            pltpu.VMEM((CH, D), jnp.float32),
            pltpu.VMEM((CH, D), jnp.float32),
            pltpu.VMEM_SHARED((NP, D), jnp.float32),
            pltpu.SemaphoreType.DMA,
            pltpu.SemaphoreType.DMA,
        ],
    )
    def prop_kernel(hs, ei3, out, ebig, r0, r1, acc, m0, m1):
        c = lax.axis_index("c")
        s = lax.axis_index("s")
        w = c * NS + s
        pltpu.sync_copy(hs.at[pl.ds(s * R, R)], acc.at[pl.ds(s * R, R)])
        plsc.subcore_barrier()

        for p in range(2):
            row0 = w * F + p * PH
            pltpu.sync_copy(ei3.at[pl.ds(row0, PH)], ebig)
            pltpu.async_copy(hs.at[ebig.at[0, 0]], r0, m0)

            @pl.loop(0, PH, step=2)
            def _(j):
                @pl.when(j + 1 < PH)
                def _():
                    pltpu.async_copy(hs.at[ebig.at[j + 1, 0]], r1, m1)

                pltpu.make_async_copy(hs.at[ebig.at[j, 0]], r0, m0).wait()
                pltpu.sync_copy(r0, acc.at[ebig.at[j, 1]], add=True)

                @pl.when(j + 2 < PH)
                def _():
                    pltpu.async_copy(hs.at[ebig.at[j + 2, 0]], r0, m0)

                pltpu.make_async_copy(hs.at[ebig.at[j + 1, 0]], r1, m1).wait()
                pltpu.sync_copy(r1, acc.at[ebig.at[j + 1, 1]], add=True)

        plsc.subcore_barrier()
        pltpu.sync_copy(acc.at[pl.ds(s * R, R)], out.at[c, pl.ds(s * R, R)])

    return prop_kernel


def _tc_first(degp, x_p, W1, NP, BM):
    nblk = NP // BM

    def body(deg_ref, x_ref, w_ref, out_ref):
        dg = deg_ref[...]
        dinv = lax.rsqrt(dg[0] + dg[1] - 1.0)
        h = jnp.dot(x_ref[...], w_ref[...], preferred_element_type=jnp.float32)
        out_ref[...] = dinv[:, None] * h

    return pl.pallas_call(
        body,
        grid=(nblk,),
        in_specs=[
            pl.BlockSpec((NC, BM), lambda i: (0, i)),
            pl.BlockSpec((BM, D), lambda i: (i, 0)),
            pl.BlockSpec((D, D), lambda i: (0, 0)),
        ],
        out_specs=pl.BlockSpec((BM, D), lambda i: (i, 0)),
        out_shape=jax.ShapeDtypeStruct((NP, D), jnp.float32),
    )(degp, x_p, W1)


def _tc_mid(a, hs_prev, degp, b_prev, W, relu, NP, BM):
    nblk = NP // BM

    def body(a_ref, hs_ref, deg_ref, b_ref, w_ref, out_ref):
        dg = deg_ref[...]
        dinv = lax.rsqrt(dg[0] + dg[1] - 1.0)
        av = a_ref[...]
        t = dinv[:, None] * (av[0] + av[1] - hs_ref[...]) + b_ref[...]
        if relu:
            t = jnp.maximum(t, 0.0)
        out_ref[...] = dinv[:, None] * jnp.dot(
            t, w_ref[...], preferred_element_type=jnp.float32)

    return pl.pallas_call(
        body,
        grid=(nblk,),
        in_specs=[
            pl.BlockSpec((NC, BM, D), lambda i: (0, i, 0)),
            pl.BlockSpec((BM, D), lambda i: (i, 0)),
            pl.BlockSpec((NC, BM), lambda i: (0, i)),
            pl.BlockSpec((1, D), lambda i: (0, 0)),
            pl.BlockSpec((D, D), lambda i: (0, 0)),
        ],
        out_specs=pl.BlockSpec((BM, D), lambda i: (i, 0)),
        out_shape=jax.ShapeDtypeStruct((NP, D), jnp.float32),
    )(a, hs_prev, degp, b_prev, W)


def _tc_pool(a, hs_prev, degp, b_prev, batch_row, NP, BM):
    nblk = NP // BM

    def body(a_ref, hs_ref, deg_ref, b_ref, bat_ref, out_ref, acc_s, acc_c):
        i = pl.program_id(0)
        dg = deg_ref[...]
        dinv = lax.rsqrt(dg[0] + dg[1] - 1.0)
        av = a_ref[...]
        x3 = dinv[:, None] * (av[0] + av[1] - hs_ref[...]) + b_ref[...]
        gid = lax.broadcasted_iota(jnp.int32, (G, 1), 0)
        pt = (bat_ref[...] == gid).astype(jnp.float32)
        part = jnp.dot(pt, x3, preferred_element_type=jnp.float32)
        cnt = jnp.broadcast_to(jnp.sum(pt, axis=1, keepdims=True), (G, D))

        @pl.when(i == 0)
        def _():
            acc_s[...] = part
            acc_c[...] = cnt

        @pl.when(i > 0)
        def _():
            acc_s[...] += part
            acc_c[...] += cnt

        @pl.when(i == nblk - 1)
        def _():
            out_ref[...] = acc_s[...] / jnp.maximum(acc_c[...], 1.0)

    return pl.pallas_call(
        body,
        grid=(nblk,),
        in_specs=[
            pl.BlockSpec((NC, BM, D), lambda i: (0, i, 0)),
            pl.BlockSpec((BM, D), lambda i: (i, 0)),
            pl.BlockSpec((NC, BM), lambda i: (0, i)),
            pl.BlockSpec((1, D), lambda i: (0, 0)),
            pl.BlockSpec((1, BM), lambda i: (0, i)),
        ],
        out_specs=pl.BlockSpec((G, D), lambda i: (0, 0)),
        out_shape=jax.ShapeDtypeStruct((G, D), jnp.float32),
        scratch_shapes=[
            pltpu.VMEM((G, D), jnp.float32),
            pltpu.VMEM((G, D), jnp.float32),
        ],
    )(a, hs_prev, degp, b_prev, batch_row)


def kernel(x, edge_index, batch, W1, b1, W2, b2, W3, b3):
    N = x.shape[0]
    E = edge_index.shape[1]
    NP = (N // 2048 + 1) * 2048
    BM = 1024 if NP % 1024 == 0 else NP // NS
    NCH0 = -(-E // CH)
    F = 4 * (-(-NCH0 // (NW * 4)))
    NCH = NW * F

    x_p = jnp.pad(x, ((0, NP - N), (0, 0)))
    ei_c = jnp.pad(edge_index, ((0, 0), (0, NCH0 * CH - E)),
                   constant_values=N)
    ei3 = jnp.transpose(ei_c.reshape(2, NCH0, CH), (1, 0, 2))
    pad_idx = (N + jnp.arange((NCH - NCH0) * CH, dtype=jnp.int32)
               % (NP - N)).reshape(NCH - NCH0, 1, CH)
    ei3p = jnp.concatenate(
        [ei3, jnp.broadcast_to(pad_idx, (NCH - NCH0, 2, CH))], axis=0)
    ones_h = jnp.ones((NP,), jnp.float32)
    batch_row = jnp.pad(batch, (0, NP - N), constant_values=G).reshape(1, NP)
    b1r, b2r, b3r = b1.reshape(1, D), b2.reshape(1, D), b3.reshape(1, D)

    degp = _make_sc_degree(NP, F)(ei3p, ones_h)
    prop = _make_sc_prop(NP, F)

    hs1 = _tc_first(degp, x_p, W1, NP, BM)
    a1 = prop(hs1, ei3p)
    hs2 = _tc_mid(a1, hs1, degp, b1r, W2, True, NP, BM)
    a2 = prop(hs2, ei3p)
    hs3 = _tc_mid(a2, hs2, degp, b2r, W3, False, NP, BM)
    a3 = prop(hs3, ei3p)
    return _tc_pool(a3, hs3, degp, b3r, batch_row, NP, BM)
